# baseline (speedup 1.0000x reference)
import functools

import jax
import jax.numpy as jnp
from jax import lax
from jax.experimental import pallas as pl
from jax.experimental.pallas import tpu as pltpu
from jax.experimental.pallas import tpu_sc as plsc

N_NODES = 10000
N_EDGES = 320000
D_FEAT = 128
RNI = 16
HIDDEN = 128
OUT = 128

NC = 2
NS = 16
NW = NC * NS
E_PER_W = N_EDGES // NW
CHUNK = 40
NCHUNK = E_PER_W // CHUNK
RING = 5
ROW_STRIDE = 624
ROW_LEN = 640
DEG_W = 128

def _deg_body(dst_hbm, ones_hbm, out_hbm, acc, ones_v, dstb, isem, ssem):
    core = lax.axis_index("c")
    sub = lax.axis_index("s")
    wid = sub * NC + core
    r0 = sub * ROW_STRIDE
    pltpu.sync_copy(ones_hbm, acc.at[pl.ds(r0, ROW_LEN)])
    pltpu.sync_copy(ones_hbm.at[pl.ds(0, CHUNK)], ones_v)
    plsc.subcore_barrier()
    base = wid * E_PER_W

    for j in range(RING):
        pltpu.async_copy(dst_hbm.at[pl.ds(base + j * CHUNK, CHUNK)],
                         dstb[j], isem[j])

    def body(t, carry):
        for j in range(RING):
            c = t * RING + j
            off = base + c * CHUNK
            pltpu.make_async_copy(dst_hbm.at[pl.ds(off, CHUNK)], dstb[j],
                                  isem[j]).wait()
            pltpu.async_copy(ones_v, acc.at[dstb[j]], ssem[j], add=True)

        @pl.when(t < NCHUNK // RING - 1)
        def _():
            for j in range(RING):
                c = t * RING + j
                pltpu.make_async_copy(ones_v, acc.at[dstb[j]], ssem[j]).wait()
                off = base + (c + RING) * CHUNK
                pltpu.async_copy(dst_hbm.at[pl.ds(off, CHUNK)], dstb[j], isem[j])

        return carry

    lax.fori_loop(0, NCHUNK // RING, body, 0)
    for j in range(RING):
        pltpu.make_async_copy(ones_v, acc.at[dstb[j]], ssem[j]).wait()
    plsc.subcore_barrier()
    pltpu.sync_copy(acc.at[pl.ds(r0, ROW_LEN)],
                    out_hbm.at[core, pl.ds(r0, ROW_LEN)])


def _msg_body(g_hbm, src_hbm, dst_hbm, out_hbm, acc,
              srcb, dstb, rows, isem, gsem, ssem):
    core = lax.axis_index("c")
    sub = lax.axis_index("s")
    wid = sub * NC + core
    r0 = sub * ROW_STRIDE
    pltpu.sync_copy(g_hbm.at[pl.ds(r0, ROW_LEN)],
                    acc.at[pl.ds(r0, ROW_LEN)])
    plsc.subcore_barrier()
    base = wid * E_PER_W

    def idx_start(c, j):
        off = base + c * CHUNK
        pltpu.async_copy(src_hbm.at[pl.ds(off, CHUNK)], srcb[j], isem[j])
        pltpu.async_copy(dst_hbm.at[pl.ds(off, CHUNK)], dstb[j], isem[j])

    def idx_wait(c, j):
        off = base + c * CHUNK
        pltpu.make_async_copy(src_hbm.at[pl.ds(off, CHUNK)], srcb[j],
                              isem[j]).wait()
        pltpu.make_async_copy(dst_hbm.at[pl.ds(off, CHUNK)], dstb[j],
                              isem[j]).wait()

    for j in range(RING):
        idx_start(j, j)

    def body(t, carry):
        for j in range(RING):
            c = t * RING + j
            idx_wait(c, j)
            pltpu.async_copy(g_hbm.at[srcb[j]], rows[j], gsem[j])
        for j in range(RING):
            pltpu.make_async_copy(g_hbm.at[srcb[j]], rows[j], gsem[j]).wait()
            pltpu.async_copy(rows[j], acc.at[dstb[j]], ssem[j], add=True)
        @pl.when(t < NCHUNK // RING - 1)
        def _():
            for j in range(RING):
                c = t * RING + j
                pltpu.make_async_copy(rows[j], acc.at[dstb[j]], ssem[j]).wait()
                idx_start(c + RING, j)

        return carry

    lax.fori_loop(0, NCHUNK // RING, body, 0)
    for j in range(RING):
        pltpu.make_async_copy(rows[j], acc.at[dstb[j]], ssem[j]).wait()
    plsc.subcore_barrier()
    pltpu.sync_copy(acc.at[pl.ds(r0, ROW_LEN)],
                    out_hbm.at[core, pl.ds(r0, ROW_LEN)])


@functools.cache
def _sc_kernels():
    mesh = plsc.VectorSubcoreMesh(core_axis_name="c", subcore_axis_name="s")
    idxbuf = [pltpu.VMEM((CHUNK,), jnp.int32) for _ in range(RING)]
    sems = lambda: [pltpu.SemaphoreType.DMA for _ in range(RING)]
    deg_pass = pl.kernel(
        _deg_body,
        out_type=jax.ShapeDtypeStruct((NC, N_NODES, DEG_W), jnp.float32),
        mesh=mesh,
        scratch_types=[
            pltpu.VMEM_SHARED((N_NODES, DEG_W), jnp.float32),
            pltpu.VMEM((CHUNK, DEG_W), jnp.float32),
            list(idxbuf), sems(), sems(),
        ],
    )
    msg_pass = pl.kernel(
        _msg_body,
        out_type=jax.ShapeDtypeStruct((NC, N_NODES, HIDDEN), jnp.float32),
        mesh=mesh,
        scratch_types=[
            pltpu.VMEM_SHARED((N_NODES, HIDDEN), jnp.float32),
            list(idxbuf), list(idxbuf),
            [pltpu.VMEM((CHUNK, HIDDEN), jnp.float32) for _ in range(RING)],
            sems(), sems(), sems(),
        ],
    )
    return deg_pass, msg_pass


def _stage0a_body(x, noise, w_top, w_bot, hw_out):
    hw_out[...] = (jnp.dot(x[...], w_top[...], preferred_element_type=jnp.float32)
                   + jnp.dot(noise[...], w_bot[...], preferred_element_type=jnp.float32))


def _stage0b_body(degp, hw, g_out, dinv_out):
    deg = degp[0, :, 0:1] + degp[1, :, 0:1] - 1.0
    dinv = jnp.where(deg > 0, lax.rsqrt(deg), 0.0)
    g_out[...] = hw[...] * dinv
    dinv_out[...] = dinv


def _stage1_body(p, g, dinv, b0, mwt, mwb, mb0, w1, g1_out):
    conv = (p[0] + p[1] - g[...]) * dinv[...] + b0[...]
    mp = jnp.max(conv, axis=0, keepdims=True)
    h1 = jnp.dot(conv, mwt[...], preferred_element_type=jnp.float32)
    h1 = h1 + jnp.dot(mp, mwb[...], preferred_element_type=jnp.float32)
    h1 = jnp.maximum(h1 + mb0[...], 0.0)
    g1_out[...] = jnp.dot(h1, w1[...], preferred_element_type=jnp.float32) * dinv[...]


def _stage2_body(p, g, dinv, b1, mwt, mwb, mb1, out):
    conv = (p[0] + p[1] - g[...]) * dinv[...] + b1[...]
    mp = jnp.max(conv, axis=0, keepdims=True)
    o = jnp.dot(conv, mwt[...], preferred_element_type=jnp.float32)
    o = o + jnp.dot(mp, mwb[...], preferred_element_type=jnp.float32)
    out[...] = o + mb1[...]


@jax.jit
def kernel(x, edge_index, orbits, noise, conv_W0, conv_b0, conv_W1, conv_b1,
           mlp_W0, mlp_b0, mlp_W1, mlp_b1):
    del orbits
    src = edge_index[0]
    dst = edge_index[1]
    ones = jnp.ones((ROW_LEN, DEG_W), jnp.float32)

    deg_pass, msg_pass = _sc_kernels()
    degp = deg_pass(dst, ones)

    hw0 = pl.pallas_call(
        _stage0a_body,
        out_shape=jax.ShapeDtypeStruct((N_NODES, HIDDEN), jnp.float32),
    )(x, noise, conv_W0[:D_FEAT], conv_W0[D_FEAT:])

    g0, dinv = pl.pallas_call(
        _stage0b_body,
        out_shape=[
            jax.ShapeDtypeStruct((N_NODES, HIDDEN), jnp.float32),
            jax.ShapeDtypeStruct((N_NODES, 1), jnp.float32),
        ],
    )(degp, hw0)

    p0 = msg_pass(g0, src, dst)

    g1 = pl.pallas_call(
        _stage1_body,
        out_shape=jax.ShapeDtypeStruct((N_NODES, HIDDEN), jnp.float32),
    )(p0, g0, dinv, conv_b0.reshape(1, HIDDEN), mlp_W0[:HIDDEN],
      mlp_W0[HIDDEN:], mlp_b0.reshape(1, HIDDEN), conv_W1)

    p1 = msg_pass(g1, src, dst)

    out = pl.pallas_call(
        _stage2_body,
        out_shape=jax.ShapeDtypeStruct((N_NODES, OUT), jnp.float32),
    )(p1, g1, dinv, conv_b1.reshape(1, HIDDEN), mlp_W1[:HIDDEN],
      mlp_W1[HIDDEN:], mlp_b1.reshape(1, OUT))

    return out


# speedup vs baseline: 1.1344x; 1.1344x over previous
import functools

import jax
import jax.numpy as jnp
from jax import lax
from jax.experimental import pallas as pl
from jax.experimental.pallas import tpu as pltpu
from jax.experimental.pallas import tpu_sc as plsc

N_NODES = 10000
N_EDGES = 320000
D_FEAT = 128
RNI = 16
HIDDEN = 128
OUT = 128

NC = 2
NS = 16
NW = NC * NS
E_PER_W = N_EDGES // NW
CHUNK = 40
NCHUNK = E_PER_W // CHUNK
RING = 5
ROW_STRIDE = 624
ROW_LEN = 640
DEG_W = 128

def _deg_body(dst_hbm, ones_hbm, out_hbm, acc, ones_v, dstb, isem, ssem):
    core = lax.axis_index("c")
    sub = lax.axis_index("s")
    wid = sub * NC + core
    r0 = sub * ROW_STRIDE
    pltpu.sync_copy(ones_hbm, acc.at[pl.ds(r0, ROW_LEN)])
    pltpu.sync_copy(ones_hbm.at[pl.ds(0, CHUNK)], ones_v)
    plsc.subcore_barrier()
    base = wid * E_PER_W

    for j in range(RING):
        pltpu.async_copy(dst_hbm.at[pl.ds(base + j * CHUNK, CHUNK)],
                         dstb[j], isem[j])

    def body(t, carry):
        for j in range(RING):
            c = t * RING + j
            off = base + c * CHUNK
            pltpu.make_async_copy(dst_hbm.at[pl.ds(off, CHUNK)], dstb[j],
                                  isem[j]).wait()
            pltpu.async_copy(ones_v, acc.at[dstb[j]], ssem[j], add=True)

        @pl.when(t < NCHUNK // RING - 1)
        def _():
            for j in range(RING):
                c = t * RING + j
                pltpu.make_async_copy(ones_v, acc.at[dstb[j]], ssem[j]).wait()
                off = base + (c + RING) * CHUNK
                pltpu.async_copy(dst_hbm.at[pl.ds(off, CHUNK)], dstb[j], isem[j])

        return carry

    lax.fori_loop(0, NCHUNK // RING, body, 0)
    for j in range(RING):
        pltpu.make_async_copy(ones_v, acc.at[dstb[j]], ssem[j]).wait()
    plsc.subcore_barrier()
    pltpu.sync_copy(acc.at[pl.ds(r0, ROW_LEN)],
                    out_hbm.at[core, pl.ds(r0, ROW_LEN)])


def _msg_body(g_hbm, src_hbm, dst_hbm, out_hbm, acc,
              srcb, dstb, rows, isem, gsem, ssem):
    core = lax.axis_index("c")
    sub = lax.axis_index("s")
    wid = sub * NC + core
    r0 = sub * ROW_STRIDE
    pltpu.sync_copy(g_hbm.at[pl.ds(r0, ROW_LEN)],
                    acc.at[pl.ds(r0, ROW_LEN)])
    plsc.subcore_barrier()
    base = wid * E_PER_W
    TMAX = NCHUNK // RING

    def idx_start(c, p, j):
        off = base + c * CHUNK
        pltpu.async_copy(src_hbm.at[pl.ds(off, CHUNK)], srcb[p][j], isem[p][j])
        pltpu.async_copy(dst_hbm.at[pl.ds(off, CHUNK)], dstb[p][j], isem[p][j])

    def idx_wait(c, p, j):
        off = base + c * CHUNK
        pltpu.make_async_copy(src_hbm.at[pl.ds(off, CHUNK)], srcb[p][j],
                              isem[p][j]).wait()
        pltpu.make_async_copy(dst_hbm.at[pl.ds(off, CHUNK)], dstb[p][j],
                              isem[p][j]).wait()

    for j in range(RING):
        idx_start(j, 0, j)

    def round_(t, p):
        for j in range(RING):
            c = t * RING + j

            @pl.when(t > 0)
            def _():
                pltpu.make_async_copy(rows[j], acc.at[dstb[1 - p][j]],
                                      ssem[j]).wait()

            @pl.when(t < TMAX - 1)
            def _():
                idx_start(c + RING, 1 - p, j)

            idx_wait(c, p, j)
            pltpu.async_copy(g_hbm.at[srcb[p][j]], rows[j], gsem[j])
        for j in range(RING):
            pltpu.make_async_copy(g_hbm.at[srcb[p][j]], rows[j], gsem[j]).wait()
            pltpu.async_copy(rows[j], acc.at[dstb[p][j]], ssem[j], add=True)

    def body(t2, carry):
        round_(2 * t2, 0)
        round_(2 * t2 + 1, 1)
        return carry

    lax.fori_loop(0, TMAX // 2, body, 0)
    for j in range(RING):
        pltpu.make_async_copy(rows[j], acc.at[dstb[1][j]], ssem[j]).wait()
    plsc.subcore_barrier()
    pltpu.sync_copy(acc.at[pl.ds(r0, ROW_LEN)],
                    out_hbm.at[core, pl.ds(r0, ROW_LEN)])


@functools.cache
def _sc_kernels():
    mesh = plsc.VectorSubcoreMesh(core_axis_name="c", subcore_axis_name="s")
    idxbuf = [pltpu.VMEM((CHUNK,), jnp.int32) for _ in range(RING)]
    sems = lambda: [pltpu.SemaphoreType.DMA for _ in range(RING)]
    deg_pass = pl.kernel(
        _deg_body,
        out_type=jax.ShapeDtypeStruct((NC, N_NODES, DEG_W), jnp.float32),
        mesh=mesh,
        scratch_types=[
            pltpu.VMEM_SHARED((N_NODES, DEG_W), jnp.float32),
            pltpu.VMEM((CHUNK, DEG_W), jnp.float32),
            list(idxbuf), sems(), sems(),
        ],
    )
    msg_pass = pl.kernel(
        _msg_body,
        out_type=jax.ShapeDtypeStruct((NC, N_NODES, HIDDEN), jnp.float32),
        mesh=mesh,
        scratch_types=[
            pltpu.VMEM_SHARED((N_NODES, HIDDEN), jnp.float32),
            [list(idxbuf), list(idxbuf)],
            [list(idxbuf), list(idxbuf)],
            [pltpu.VMEM((CHUNK, HIDDEN), jnp.float32) for _ in range(RING)],
            [sems(), sems()], sems(), sems(),
        ],
    )
    return deg_pass, msg_pass


def _stage0a_body(x, noise, w_top, w_bot, hw_out):
    hw_out[...] = (jnp.dot(x[...], w_top[...], preferred_element_type=jnp.float32)
                   + jnp.dot(noise[...], w_bot[...], preferred_element_type=jnp.float32))


def _stage0b_body(degp, hw, g_out, dinv_out):
    deg = degp[0, :, 0:1] + degp[1, :, 0:1] - 1.0
    dinv = jnp.where(deg > 0, lax.rsqrt(deg), 0.0)
    g_out[...] = hw[...] * dinv
    dinv_out[...] = dinv


def _stage1_body(p, g, dinv, b0, mwt, mwb, mb0, w1, g1_out):
    conv = (p[0] + p[1] - g[...]) * dinv[...] + b0[...]
    mp = jnp.max(conv, axis=0, keepdims=True)
    h1 = jnp.dot(conv, mwt[...], preferred_element_type=jnp.float32)
    h1 = h1 + jnp.dot(mp, mwb[...], preferred_element_type=jnp.float32)
    h1 = jnp.maximum(h1 + mb0[...], 0.0)
    g1_out[...] = jnp.dot(h1, w1[...], preferred_element_type=jnp.float32) * dinv[...]


def _stage2_body(p, g, dinv, b1, mwt, mwb, mb1, out):
    conv = (p[0] + p[1] - g[...]) * dinv[...] + b1[...]
    mp = jnp.max(conv, axis=0, keepdims=True)
    o = jnp.dot(conv, mwt[...], preferred_element_type=jnp.float32)
    o = o + jnp.dot(mp, mwb[...], preferred_element_type=jnp.float32)
    out[...] = o + mb1[...]


@jax.jit
def kernel(x, edge_index, orbits, noise, conv_W0, conv_b0, conv_W1, conv_b1,
           mlp_W0, mlp_b0, mlp_W1, mlp_b1):
    del orbits
    src = edge_index[0]
    dst = edge_index[1]
    ones = jnp.ones((ROW_LEN, DEG_W), jnp.float32)

    deg_pass, msg_pass = _sc_kernels()
    degp = deg_pass(dst, ones)

    hw0 = pl.pallas_call(
        _stage0a_body,
        out_shape=jax.ShapeDtypeStruct((N_NODES, HIDDEN), jnp.float32),
    )(x, noise, conv_W0[:D_FEAT], conv_W0[D_FEAT:])

    g0, dinv = pl.pallas_call(
        _stage0b_body,
        out_shape=[
            jax.ShapeDtypeStruct((N_NODES, HIDDEN), jnp.float32),
            jax.ShapeDtypeStruct((N_NODES, 1), jnp.float32),
        ],
    )(degp, hw0)

    p0 = msg_pass(g0, src, dst)

    g1 = pl.pallas_call(
        _stage1_body,
        out_shape=jax.ShapeDtypeStruct((N_NODES, HIDDEN), jnp.float32),
    )(p0, g0, dinv, conv_b0.reshape(1, HIDDEN), mlp_W0[:HIDDEN],
      mlp_W0[HIDDEN:], mlp_b0.reshape(1, HIDDEN), conv_W1)

    p1 = msg_pass(g1, src, dst)

    out = pl.pallas_call(
        _stage2_body,
        out_shape=jax.ShapeDtypeStruct((N_NODES, OUT), jnp.float32),
    )(p1, g1, dinv, conv_b1.reshape(1, HIDDEN), mlp_W1[:HIDDEN],
      mlp_W1[HIDDEN:], mlp_b1.reshape(1, OUT))

    return out
